# Initial kernel scaffold; baseline (speedup 1.0000x reference)
import jax
import jax.numpy as jnp
from jax.experimental import pallas as pl


def kernel(features, OD):
    raise NotImplementedError("write your pallas kernel here")


# speedup vs baseline: 1.8195x; 1.8195x over previous
import jax
import jax.numpy as jnp
from jax.experimental import pallas as pl
from jax.experimental.pallas import tpu as pltpu

_BLOCK_ROWS = 256
_ROW = 2048


def _softmax_block(f_ref, od_ref, out_ref):
    f = f_ref[...]
    od = od_ref[...]
    mask = jnp.logical_and(od != 0, f != 0.0)
    neg_inf = jnp.float32(-jnp.inf)
    v = jnp.where(mask, f, neg_inf)
    mx = jnp.max(v, axis=-1, keepdims=True)
    mx = jnp.where(jnp.isfinite(mx), mx, jnp.zeros_like(mx))
    e = jnp.where(mask, jnp.exp(f - mx), 0.0)
    s = jnp.sum(e, axis=-1, keepdims=True)
    s = jnp.where(s == 0.0, jnp.ones_like(s), s)
    out_ref[...] = e / s


def kernel(features, OD):
    shape = features.shape
    rows = 1
    for d in shape[:-1]:
        rows *= d
    f2 = features.reshape(rows, shape[-1])
    od2 = OD.reshape(rows, shape[-1])
    grid = (rows // _BLOCK_ROWS,)
    out = pl.pallas_call(
        _softmax_block,
        grid=grid,
        in_specs=[
            pl.BlockSpec((_BLOCK_ROWS, _ROW), lambda i: (i, 0)),
            pl.BlockSpec((_BLOCK_ROWS, _ROW), lambda i: (i, 0)),
        ],
        out_specs=pl.BlockSpec((_BLOCK_ROWS, _ROW), lambda i: (i, 0)),
        out_shape=jax.ShapeDtypeStruct((rows, shape[-1]), features.dtype),
        compiler_params=pltpu.CompilerParams(
            dimension_semantics=("arbitrary",),
        ),
    )(f2, od2)
    return out.reshape(shape)


# speedup vs baseline: 1.8874x; 1.0373x over previous
import jax
import jax.numpy as jnp
from jax.experimental import pallas as pl
from jax.experimental.pallas import tpu as pltpu

_BLOCK_ROWS = 512
_ROW = 2048


def _softmax_block(f_ref, od_ref, out_ref):
    f = f_ref[...]
    od = od_ref[...]
    mask = jnp.logical_and(od != 0, f != 0.0)
    neg_inf = jnp.float32(-jnp.inf)
    v = jnp.where(mask, f, neg_inf)
    mx = jnp.max(v, axis=-1, keepdims=True)
    mx = jnp.where(jnp.isfinite(mx), mx, jnp.zeros_like(mx))
    e = jnp.where(mask, jnp.exp(f - mx), 0.0)
    s = jnp.sum(e, axis=-1, keepdims=True)
    s = jnp.where(s == 0.0, jnp.ones_like(s), s)
    out_ref[...] = e / s


def kernel(features, OD):
    shape = features.shape
    rows = 1
    for d in shape[:-1]:
        rows *= d
    f2 = features.reshape(rows, shape[-1])
    od2 = OD.reshape(rows, shape[-1])
    grid = (rows // _BLOCK_ROWS,)
    out = pl.pallas_call(
        _softmax_block,
        grid=grid,
        in_specs=[
            pl.BlockSpec((_BLOCK_ROWS, _ROW), lambda i: (i, 0)),
            pl.BlockSpec((_BLOCK_ROWS, _ROW), lambda i: (i, 0)),
        ],
        out_specs=pl.BlockSpec((_BLOCK_ROWS, _ROW), lambda i: (i, 0)),
        out_shape=jax.ShapeDtypeStruct((rows, shape[-1]), features.dtype),
        compiler_params=pltpu.CompilerParams(
            dimension_semantics=("arbitrary",),
        ),
    )(f2, od2)
    return out.reshape(shape)


# speedup vs baseline: 1.9890x; 1.0538x over previous
import jax
import jax.numpy as jnp
from jax.experimental import pallas as pl
from jax.experimental.pallas import tpu as pltpu

_BLOCK_ROWS = 1024
_ROW = 2048


def _softmax_block(f_ref, od_ref, out_ref):
    f = f_ref[...]
    od = od_ref[...]
    mask = jnp.logical_and(od != 0, f != 0.0)
    neg_inf = jnp.float32(-jnp.inf)
    v = jnp.where(mask, f, neg_inf)
    mx = jnp.max(v, axis=-1, keepdims=True)
    mx = jnp.where(jnp.isfinite(mx), mx, jnp.zeros_like(mx))
    e = jnp.where(mask, jnp.exp(f - mx), 0.0)
    s = jnp.sum(e, axis=-1, keepdims=True)
    s = jnp.where(s == 0.0, jnp.ones_like(s), s)
    out_ref[...] = e / s


def kernel(features, OD):
    shape = features.shape
    rows = 1
    for d in shape[:-1]:
        rows *= d
    f2 = features.reshape(rows, shape[-1])
    od2 = OD.reshape(rows, shape[-1])
    grid = (rows // _BLOCK_ROWS,)
    out = pl.pallas_call(
        _softmax_block,
        grid=grid,
        in_specs=[
            pl.BlockSpec((_BLOCK_ROWS, _ROW), lambda i: (i, 0)),
            pl.BlockSpec((_BLOCK_ROWS, _ROW), lambda i: (i, 0)),
        ],
        out_specs=pl.BlockSpec((_BLOCK_ROWS, _ROW), lambda i: (i, 0)),
        out_shape=jax.ShapeDtypeStruct((rows, shape[-1]), features.dtype),
        compiler_params=pltpu.CompilerParams(
            dimension_semantics=("arbitrary",),
        ),
    )(f2, od2)
    return out.reshape(shape)
